# Initial kernel scaffold; baseline (speedup 1.0000x reference)
import jax
import jax.numpy as jnp
from jax.experimental import pallas as pl


def kernel(x, edge_index, head, Wg1_1, Wg2_1, Wb1_1, Wb2_1, r_1, W_1, b_1, Wg1_2, Wg2_2, Wb1_2, Wb2_2, r_2, W_2, b_2):
    raise NotImplementedError("write your pallas kernel here")


# speedup vs baseline: 11.4311x; 11.4311x over previous
import functools

import jax
import jax.numpy as jnp
from jax import lax
from jax.experimental import pallas as pl
from jax.experimental.pallas import tpu as pltpu
from jax.experimental.pallas import tpu_sc as plsc

N = 10000
FEAT, HID, NCLS = 128, 32, 16
E = 320000
NC, NS = 2, 16
NW = NC * NS
CHUNK = 128
CPW = 79
EPAD = NW * CPW * CHUNK
ZPS = 632
NPAD = NS * ZPS
CW = 8


def _seg_mesh():
    return plsc.VectorSubcoreMesh(core_axis_name="c", subcore_axis_name="s",
                                  num_cores=NC, num_subcores=NS)


def _seg_sum_count(table, colT, rowT):
    feat = table.shape[1]

    @functools.partial(
        pl.kernel,
        out_type=(jax.ShapeDtypeStruct((NC * NPAD, feat), jnp.float32),
                  jax.ShapeDtypeStruct((NC * NPAD, CW), jnp.float32)),
        mesh=_seg_mesh(),
        compiler_params=pltpu.CompilerParams(use_tc_tiling_on_sc=False),
        scratch_types=[
            pltpu.VMEM((1, CHUNK), jnp.int32),
            pltpu.VMEM((CPW, CHUNK), jnp.int32),
            pltpu.VMEM((CHUNK, feat), jnp.float32),
            pltpu.VMEM((CHUNK, CW), jnp.float32),
            pltpu.VMEM_SHARED((NPAD, feat), jnp.float32),
            pltpu.VMEM_SHARED((NPAD, CW), jnp.float32),
            pltpu.SemaphoreType.DMA,
        ],
    )
    def body(table_h, colT_h, rowT_h, zf_h, zc_h, ones_h,
             s_out, c_out, colv, rowv, rowsv, onesv, accS, accC, sem):
        c = lax.axis_index("c")
        s = lax.axis_index("s")
        wid = c * NS + s
        pltpu.sync_copy(zf_h, accS.at[pl.ds(s * ZPS, ZPS)])
        pltpu.sync_copy(zc_h, accC.at[pl.ds(s * ZPS, ZPS)])
        pltpu.sync_copy(ones_h, onesv)
        pltpu.sync_copy(rowT_h.at[wid], rowv)
        plsc.subcore_barrier()

        def step(j, carry):
            pltpu.sync_copy(colT_h.at[wid, j], colv.at[0])
            pltpu.async_copy(table_h.at[colv.at[0]], rowsv, sem).wait()
            pltpu.sync_copy(rowsv, accS.at[rowv.at[j]], add=True)
            pltpu.sync_copy(onesv, accC.at[rowv.at[j]], add=True)
            return carry

        lax.fori_loop(0, CPW, step, 0)
        plsc.subcore_barrier()
        off = c * NPAD + s * ZPS
        pltpu.sync_copy(accS.at[pl.ds(s * ZPS, ZPS)], s_out.at[pl.ds(off, ZPS)])
        pltpu.sync_copy(accC.at[pl.ds(s * ZPS, ZPS)], c_out.at[pl.ds(off, ZPS)])

    zf = jnp.zeros((ZPS, feat), jnp.float32)
    zc = jnp.zeros((ZPS, CW), jnp.float32)
    ones = jnp.ones((CHUNK, CW), jnp.float32)
    return body(table, colT, rowT, zf, zc, ones)


def _seg_sum(table, colT, rowT):
    feat = table.shape[1]

    @functools.partial(
        pl.kernel,
        out_type=jax.ShapeDtypeStruct((NC * NPAD, feat), jnp.float32),
        mesh=_seg_mesh(),
        compiler_params=pltpu.CompilerParams(use_tc_tiling_on_sc=False),
        scratch_types=[
            pltpu.VMEM((1, CHUNK), jnp.int32),
            pltpu.VMEM((CPW, CHUNK), jnp.int32),
            pltpu.VMEM((CHUNK, feat), jnp.float32),
            pltpu.VMEM_SHARED((NPAD, feat), jnp.float32),
            pltpu.SemaphoreType.DMA,
        ],
    )
    def body(table_h, colT_h, rowT_h, zf_h,
             s_out, colv, rowv, rowsv, accS, sem):
        c = lax.axis_index("c")
        s = lax.axis_index("s")
        wid = c * NS + s
        pltpu.sync_copy(zf_h, accS.at[pl.ds(s * ZPS, ZPS)])
        pltpu.sync_copy(rowT_h.at[wid], rowv)
        plsc.subcore_barrier()

        def step(j, carry):
            pltpu.sync_copy(colT_h.at[wid, j], colv.at[0])
            pltpu.async_copy(table_h.at[colv.at[0]], rowsv, sem).wait()
            pltpu.sync_copy(rowsv, accS.at[rowv.at[j]], add=True)
            return carry

        lax.fori_loop(0, CPW, step, 0)
        plsc.subcore_barrier()
        off = c * NPAD + s * ZPS
        pltpu.sync_copy(accS.at[pl.ds(s * ZPS, ZPS)], s_out.at[pl.ds(off, ZPS)])

    zf = jnp.zeros((ZPS, feat), jnp.float32)
    return body(table, colT, rowT, zf)


def _lrelu(v):
    return jnp.where(v >= 0, v, 0.2 * v)


BM = 1000


def _dot(a, b):
    return jnp.dot(a, b, preferred_element_type=jnp.float32)


def _tc1_body(x, s0, s1, c0, c1, wg1t, wg2t, wb1t, wb2t, r1, w1,
              out1, xw_o, y1_o):
    cc = c0[:, :1] + c1[:, :1]
    nb = (s0[:, :] + s1[:, :]) / jnp.maximum(cc, 1.0)
    dinv = lax.rsqrt(cc + 1.0)
    xv = x[:, :]
    g = _lrelu(_dot(xv, wg1t[:, :]) + _dot(nb, wg2t[:, :])) + 1.0
    b = _lrelu(_dot(xv, wb1t[:, :]) + _dot(nb, wb2t[:, :]))
    out1[:, :] = xv + g * r1[:, :] + b - nb
    xw = _dot(xv, w1[:, :])
    xw_o[:, :] = xw
    y1_o[:, :] = dinv * xw


def _tc2_body(sy0, sy1, xw, c0, c1, b1, w2, x1_o, xw2_o, y2_o):
    cc = c0[:, :1] + c1[:, :1]
    dinv = lax.rsqrt(cc + 1.0)
    h1 = dinv * (sy0[:, :] + sy1[:, :]) + dinv * dinv * xw[:, :] + b1[:, :]
    x1 = jnp.where(h1 > 0, h1, jnp.exp(h1) - 1.0)
    x1_o[:, :] = x1
    xw2 = _dot(x1, w2[:, :])
    xw2_o[:, :] = xw2
    y2_o[:, :] = dinv * xw2


def _tc3_body(sn0, sn1, sy0, sy1, x1, xw2, c0, c1,
              wg1t, wg2t, wb1t, wb2t, r2, b2, out2, h2_o, logp_o):
    cc = c0[:, :1] + c1[:, :1]
    dinv = lax.rsqrt(cc + 1.0)
    nb = (sn0[:, :] + sn1[:, :]) / jnp.maximum(cc, 1.0)
    x1v = x1[:, :]
    g = _lrelu(_dot(x1v, wg1t[:, :]) + _dot(nb, wg2t[:, :])) + 1.0
    b = _lrelu(_dot(x1v, wb1t[:, :]) + _dot(nb, wb2t[:, :]))
    out2[:, :] = x1v + g * r2[:, :] + b - nb
    h2 = dinv * (sy0[:, :] + sy1[:, :]) + dinv * dinv * xw2[:, :] + b2[:, :]
    h2_o[:, :] = h2
    m = jnp.max(h2, axis=1, keepdims=True)
    sh = h2 - m
    logp_o[:, :] = sh - jnp.log(jnp.sum(jnp.exp(sh), axis=1, keepdims=True))


def _row_spec(w):
    return pl.BlockSpec((BM, w), lambda i: (i, 0))


def _full_spec(h, w):
    return pl.BlockSpec((h, w), lambda i: (0, 0))


def _f32(shape):
    return jax.ShapeDtypeStruct(shape, jnp.float32)


def kernel(x, edge_index, head, Wg1_1, Wg2_1, Wb1_1, Wb2_1, r_1, W_1, b_1,
           Wg1_2, Wg2_2, Wb1_2, Wb2_2, r_2, W_2, b_2):
    del head
    grid = (N // BM,)
    row, col = edge_index[0], edge_index[1]
    padE = EPAD - E
    rowp = jnp.concatenate([row, jnp.full((padE,), N, jnp.int32)])
    colp = jnp.concatenate([col, jnp.zeros((padE,), jnp.int32)])
    rowT = rowp.reshape(NW, CPW, CHUNK)
    colT = colp.reshape(NW, CPW, CHUNK)

    S, C = _seg_sum_count(x, colT, rowT)
    s0, s1 = S[:N], S[NPAD:NPAD + N]
    c0, c1 = C[:N], C[NPAD:NPAD + N]

    tc1 = pl.pallas_call(
        _tc1_body,
        grid=grid,
        in_specs=[_row_spec(FEAT), _row_spec(FEAT), _row_spec(FEAT),
                  _row_spec(CW), _row_spec(CW),
                  _full_spec(FEAT, FEAT), _full_spec(FEAT, FEAT),
                  _full_spec(FEAT, FEAT), _full_spec(FEAT, FEAT),
                  _full_spec(1, FEAT), _full_spec(FEAT, HID)],
        out_specs=[_row_spec(FEAT), _row_spec(HID), _row_spec(HID)],
        out_shape=[_f32((N, FEAT)), _f32((N, HID)), _f32((N, HID))],
    )
    out1, xw, y1 = tc1(x, s0, s1, c0, c1, Wg1_1.T, Wg2_1.T, Wb1_1.T, Wb2_1.T,
                       r_1, W_1)

    Sy = _seg_sum(y1, colT, rowT)

    tc2 = pl.pallas_call(
        _tc2_body,
        grid=grid,
        in_specs=[_row_spec(HID), _row_spec(HID), _row_spec(HID),
                  _row_spec(CW), _row_spec(CW),
                  _full_spec(1, HID), _full_spec(HID, NCLS)],
        out_specs=[_row_spec(HID), _row_spec(NCLS), _row_spec(NCLS)],
        out_shape=[_f32((N, HID)), _f32((N, NCLS)), _f32((N, NCLS))],
    )
    x1, xw2, y2 = tc2(Sy[:N], Sy[NPAD:NPAD + N], xw, c0, c1, b_1.reshape(1, HID), W_2)

    z2 = jnp.concatenate([x1, y2], axis=1)
    Sz = _seg_sum(z2, colT, rowT)
    Sz3 = Sz.reshape(NC, NPAD, HID + NCLS)
    sn0, sn1 = Sz3[0, :N, :HID], Sz3[1, :N, :HID]
    sy0, sy1 = Sz3[0, :N, HID:], Sz3[1, :N, HID:]

    tc3 = pl.pallas_call(
        _tc3_body,
        grid=grid,
        in_specs=[_row_spec(HID), _row_spec(HID), _row_spec(NCLS),
                  _row_spec(NCLS), _row_spec(HID), _row_spec(NCLS),
                  _row_spec(CW), _row_spec(CW),
                  _full_spec(HID, HID), _full_spec(HID, HID),
                  _full_spec(HID, HID), _full_spec(HID, HID),
                  _full_spec(1, HID), _full_spec(1, NCLS)],
        out_specs=[_row_spec(HID), _row_spec(NCLS), _row_spec(NCLS)],
        out_shape=[_f32((N, HID)), _f32((N, NCLS)), _f32((N, NCLS))],
    )
    out2, h2, logp = tc3(sn0, sn1, sy0, sy1, x1, xw2, c0, c1,
                         Wg1_2.T, Wg2_2.T, Wb1_2.T, Wb2_2.T,
                         r_2, b_2.reshape(1, NCLS))
    return (h2, logp, out1, out2)
